# Initial kernel scaffold; baseline (speedup 1.0000x reference)
import jax
import jax.numpy as jnp
from jax.experimental import pallas as pl


def kernel(x, r_ij, neighbors_j, neighbors_k, pairwise_mask, W1f, b1f, W2f, b2f, W_in2f, W_f2out, b_f2out):
    raise NotImplementedError("write your pallas kernel here")


# speedup vs baseline: 13.5736x; 13.5736x over previous
import functools
import math

import jax
import jax.numpy as jnp
from jax import lax
from jax.experimental import pallas as pl
from jax.experimental.pallas import tpu as pltpu
from jax.experimental.pallas import tpu_sc as plsc

_LOG2 = math.log(2.0)

B, A, P, G, F = 8, 1024, 32, 16, 128
NPAIR = B * A * P
NW = 32
CH = 128
CHUNKS_PER_W = NPAIR // (NW * CH)
NROW = NPAIR // CH


def _ssp(u):
    return jnp.maximum(u, 0.0) + jnp.log1p(jnp.exp(-jnp.abs(u))) - _LOG2


def _in2f_body(x_ref, w_ref, y_ref):
    y_ref[...] = jnp.dot(x_ref[...], w_ref[...],
                         preferred_element_type=jnp.float32)


def _in2f(x2, w):
    return pl.pallas_call(
        _in2f_body,
        grid=(B,),
        in_specs=[
            pl.BlockSpec((A, F), lambda i: (i, 0)),
            pl.BlockSpec((F, F), lambda i: (0, 0)),
        ],
        out_specs=pl.BlockSpec((A, F), lambda i: (i, 0)),
        out_shape=jax.ShapeDtypeStruct((B * A, F), jnp.float32),
    )(x2, w)


def _sc_gather_mul(y2, idx_j, idx_k):
    mesh = plsc.VectorSubcoreMesh(core_axis_name="c", subcore_axis_name="s")

    @functools.partial(
        pl.kernel,
        out_type=jax.ShapeDtypeStruct((NROW, CH, F), jnp.float32),
        mesh=mesh,
        scratch_types=[
            pltpu.VMEM((CH,), jnp.int32),
            pltpu.VMEM((CH,), jnp.int32),
            pltpu.VMEM((CH, F), jnp.float32),
            pltpu.VMEM((CH, F), jnp.float32),
            pltpu.SemaphoreType.DMA,
            pltpu.SemaphoreType.DMA,
        ],
    )
    def body(y2_hbm, ij_hbm, ik_hbm, g_hbm, ij_v, ik_v, rj_v, rk_v, s1, s2):
        wid = lax.axis_index("s") * 2 + lax.axis_index("c")

        def chunk(c, carry):
            row = wid * CHUNKS_PER_W + c
            pltpu.sync_copy(ij_hbm.at[row], ij_v)
            pltpu.sync_copy(ik_hbm.at[row], ik_v)
            cj = pltpu.async_copy(y2_hbm.at[ij_v], rj_v, s1)
            ck = pltpu.async_copy(y2_hbm.at[ik_v], rk_v, s2)
            cj.wait()
            ck.wait()

            def mulrow(r, carry2):
                for cc in range(F // 16):
                    sl = pl.ds(cc * 16, 16)
                    rj_v[r, sl] = rj_v[r, sl] * rk_v[r, sl]
                return carry2

            lax.fori_loop(0, CH, mulrow, 0, unroll=2)
            pltpu.sync_copy(rj_v, g_hbm.at[row])
            return carry

        lax.fori_loop(0, CHUNKS_PER_W, chunk, 0)

    return body(y2, idx_j, idx_k)


def _tc2_body(r2_ref, g_ref, m_ref, w1_ref, b1_ref, w2_ref, b2_ref,
              wo_ref, bo_ref, out_ref, *, ablk):
    r2 = r2_ref[0]
    h = jnp.dot(r2, w1_ref[...], preferred_element_type=jnp.float32)
    h = _ssp(h + b1_ref[...])
    w = jnp.dot(h, w2_ref[...], preferred_element_type=jnp.float32)
    w = w + b2_ref[...]
    wg = w * g_ref[...]
    wg = wg.reshape(ablk, P, F) * m_ref[0][:, :, None]
    agg = jnp.sum(wg, axis=1)
    out = jnp.dot(agg, wo_ref[...], preferred_element_type=jnp.float32)
    out_ref[0] = _ssp(out + bo_ref[...])


def _tc2(r2, g2, mask, w1, b1, w2, b2, wo, bo, ablk=256):
    nblk = A // ablk
    bp = ablk * P
    body = functools.partial(_tc2_body, ablk=ablk)
    return pl.pallas_call(
        body,
        grid=(B, nblk),
        in_specs=[
            pl.BlockSpec((1, bp, G), lambda i, j: (i, j, 0)),
            pl.BlockSpec((bp, F), lambda i, j: (i * nblk + j, 0)),
            pl.BlockSpec((1, ablk, P), lambda i, j: (i, j, 0)),
            pl.BlockSpec((G, F), lambda i, j: (0, 0)),
            pl.BlockSpec((1, F), lambda i, j: (0, 0)),
            pl.BlockSpec((F, F), lambda i, j: (0, 0)),
            pl.BlockSpec((1, F), lambda i, j: (0, 0)),
            pl.BlockSpec((F, F), lambda i, j: (0, 0)),
            pl.BlockSpec((1, F), lambda i, j: (0, 0)),
        ],
        out_specs=pl.BlockSpec((1, ablk, F), lambda i, j: (i, j, 0)),
        out_shape=jax.ShapeDtypeStruct((B, A, F), jnp.float32),
    )(r2, g2, mask, w1, b1, w2, b2, wo, bo)


def kernel(x, r_ij, neighbors_j, neighbors_k, pairwise_mask,
           W1f, b1f, W2f, b2f, W_in2f, W_f2out, b_f2out):
    x2 = x.reshape(B * A, F)
    y2 = _in2f(x2, W_in2f)

    boff = (jnp.arange(B, dtype=jnp.int32) * A)[:, None, None]
    ij = (neighbors_j + boff).reshape(NROW, CH)
    ik = (neighbors_k + boff).reshape(NROW, CH)

    g = _sc_gather_mul(y2, ij, ik)
    g2 = g.reshape(NPAIR, F)

    r2 = r_ij.reshape(B, A * P, G)
    out = _tc2(r2, g2, pairwise_mask,
               W1f, b1f.reshape(1, F), W2f, b2f.reshape(1, F),
               W_f2out, b_f2out.reshape(1, F))
    return out


# speedup vs baseline: 18.5764x; 1.3686x over previous
import functools
import math

import jax
import jax.numpy as jnp
from jax import lax
from jax.experimental import pallas as pl
from jax.experimental.pallas import tpu as pltpu
from jax.experimental.pallas import tpu_sc as plsc

_LOG2 = math.log(2.0)

B, A, P, G, F = 8, 1024, 32, 16, 128
NPAIR = B * A * P
NW = 32
CH = 128
CHUNKS_PER_W = NPAIR // (NW * CH)
NROW = NPAIR // CH


def _ssp(u):
    return jnp.maximum(u, 0.0) + jnp.log1p(jnp.exp(-jnp.abs(u))) - _LOG2


def _in2f_body(x_ref, w_ref, y_ref):
    y_ref[...] = jnp.dot(x_ref[...], w_ref[...],
                         preferred_element_type=jnp.float32)


def _in2f(x2, w):
    return pl.pallas_call(
        _in2f_body,
        grid=(B,),
        in_specs=[
            pl.BlockSpec((A, F), lambda i: (i, 0)),
            pl.BlockSpec((F, F), lambda i: (0, 0)),
        ],
        out_specs=pl.BlockSpec((A, F), lambda i: (i, 0)),
        out_shape=jax.ShapeDtypeStruct((B * A, F), jnp.float32),
    )(x2, w)


def _sc_gather_mul(y2, idx_j, idx_k):
    mesh = plsc.VectorSubcoreMesh(core_axis_name="c", subcore_axis_name="s")

    nbuf = 2
    cw = CHUNKS_PER_W

    @functools.partial(
        pl.kernel,
        out_type=jax.ShapeDtypeStruct((NROW, CH, F), jnp.float32),
        mesh=mesh,
        scratch_types=[
            pltpu.VMEM((cw, CH), jnp.int32),
            pltpu.VMEM((cw, CH), jnp.int32),
            [pltpu.VMEM((CH, F), jnp.float32)] * nbuf,
            [pltpu.VMEM((CH, F), jnp.float32)] * nbuf,
            [pltpu.SemaphoreType.DMA] * nbuf,
            pltpu.SemaphoreType.DMA,
        ],
    )
    def body(y2_hbm, ij_hbm, ik_hbm, g_hbm,
             ij_v, ik_v, rjs, rks, sems, s_ix):
        wid = lax.axis_index("s") * 2 + lax.axis_index("c")
        row0 = wid * cw
        ci = pltpu.async_copy(ij_hbm.at[pl.ds(row0, cw)], ij_v, s_ix)
        ck = pltpu.async_copy(ik_hbm.at[pl.ds(row0, cw)], ik_v, s_ix)
        ci.wait()
        ck.wait()

        def gather(c, b):
            pltpu.async_copy(y2_hbm.at[ij_v.at[c]], rjs[b], sems[b])
            pltpu.async_copy(y2_hbm.at[ik_v.at[c]], rks[b], sems[b])

        def drain(c, b):
            pltpu.make_async_copy(y2_hbm.at[ij_v.at[c]], rjs[b], sems[b]).wait()
            pltpu.make_async_copy(y2_hbm.at[ik_v.at[c]], rks[b], sems[b]).wait()

        def process(c, b):
            drain(c, b)
            rj_v, rk_v = rjs[b], rks[b]

            def mulrow(r, carry2):
                for cc in range(F // 16):
                    sl = pl.ds(cc * 16, 16)
                    rj_v[r, sl] = rj_v[r, sl] * rk_v[r, sl]
                return carry2

            lax.fori_loop(0, CH, mulrow, 0, unroll=2)
            pltpu.sync_copy(rj_v, g_hbm.at[row0 + c])

        gather(0, 0)

        def step(t, carry):
            c0 = 2 * t
            gather(c0 + 1, 1)
            process(c0, 0)
            nxt = jnp.minimum(c0 + 2, cw - 2)
            gather(nxt, 0)
            process(c0 + 1, 1)
            return carry

        lax.fori_loop(0, cw // 2, step, 0)
        drain(cw - 2, 0)

    return body(y2, idx_j, idx_k)


def _tc2_body(r2_ref, g_ref, m_ref, w1_ref, b1_ref, w2_ref, b2_ref,
              wo_ref, bo_ref, out_ref, *, ablk):
    r2 = r2_ref[0]
    h = jnp.dot(r2, w1_ref[...], preferred_element_type=jnp.float32)
    h = _ssp(h + b1_ref[...])
    w = jnp.dot(h, w2_ref[...], preferred_element_type=jnp.float32)
    w = w + b2_ref[...]
    wg = w * g_ref[...]
    wg = wg.reshape(ablk, P, F) * m_ref[0][:, :, None]
    agg = jnp.sum(wg, axis=1)
    out = jnp.dot(agg, wo_ref[...], preferred_element_type=jnp.float32)
    out_ref[0] = _ssp(out + bo_ref[...])


def _tc2(r2, g2, mask, w1, b1, w2, b2, wo, bo, ablk=256):
    nblk = A // ablk
    bp = ablk * P
    body = functools.partial(_tc2_body, ablk=ablk)
    return pl.pallas_call(
        body,
        grid=(B, nblk),
        in_specs=[
            pl.BlockSpec((1, bp, G), lambda i, j: (i, j, 0)),
            pl.BlockSpec((bp, F), lambda i, j: (i * nblk + j, 0)),
            pl.BlockSpec((1, ablk, P), lambda i, j: (i, j, 0)),
            pl.BlockSpec((G, F), lambda i, j: (0, 0)),
            pl.BlockSpec((1, F), lambda i, j: (0, 0)),
            pl.BlockSpec((F, F), lambda i, j: (0, 0)),
            pl.BlockSpec((1, F), lambda i, j: (0, 0)),
            pl.BlockSpec((F, F), lambda i, j: (0, 0)),
            pl.BlockSpec((1, F), lambda i, j: (0, 0)),
        ],
        out_specs=pl.BlockSpec((1, ablk, F), lambda i, j: (i, j, 0)),
        out_shape=jax.ShapeDtypeStruct((B, A, F), jnp.float32),
    )(r2, g2, mask, w1, b1, w2, b2, wo, bo)


def kernel(x, r_ij, neighbors_j, neighbors_k, pairwise_mask,
           W1f, b1f, W2f, b2f, W_in2f, W_f2out, b_f2out):
    x2 = x.reshape(B * A, F)
    y2 = _in2f(x2, W_in2f)

    boff = (jnp.arange(B, dtype=jnp.int32) * A)[:, None, None]
    ij = (neighbors_j + boff).reshape(NROW, CH)
    ik = (neighbors_k + boff).reshape(NROW, CH)

    g = _sc_gather_mul(y2, ij, ik)
    g2 = g.reshape(NPAIR, F)

    r2 = r_ij.reshape(B, A * P, G)
    out = _tc2(r2, g2, pairwise_mask,
               W1f, b1f.reshape(1, F), W2f, b2f.reshape(1, F),
               W_f2out, b_f2out.reshape(1, F))
    return out
